# Initial kernel scaffold; baseline (speedup 1.0000x reference)
import jax
import jax.numpy as jnp
from jax.experimental import pallas as pl


def kernel(z, label, table):
    raise NotImplementedError("write your pallas kernel here")


# speedup vs baseline: 1.8074x; 1.8074x over previous
import functools

import jax
import jax.numpy as jnp
from jax import lax
from jax.experimental import pallas as pl
from jax.experimental.pallas import tpu as pltpu
from jax.experimental.pallas import tpu_sc as plsc

LATENT = 128
BATCH = 16384
NC, NS, L = 2, 16, 16
NW = NC * NS
BPW = BATCH // NW
CH = 128
NCHUNK = BPW // CH

_mesh = plsc.VectorSubcoreMesh(core_axis_name="c", subcore_axis_name="s")


@functools.partial(
    pl.kernel,
    mesh=_mesh,
    out_type=jax.ShapeDtypeStruct((BATCH, LATENT), jnp.float32),
    scratch_types=[
        pltpu.VMEM((BPW,), jnp.int32),
        pltpu.VMEM((CH, LATENT), jnp.float32),
        pltpu.VMEM((CH, LATENT), jnp.float32),
        pltpu.SemaphoreType.DMA,
    ],
)
def _emb_mul(z_hbm, label_hbm, table_hbm, out_hbm, idx_v, z_v, rows_v, sem):
    wid = lax.axis_index("s") * NC + lax.axis_index("c")
    base = wid * BPW
    pltpu.sync_copy(label_hbm.at[pl.ds(base, BPW)], idx_v)
    for c in range(NCHUNK):
        off = base + c * CH
        gather = pltpu.async_copy(
            table_hbm.at[idx_v.at[pl.ds(c * CH, CH)]], rows_v, sem)
        pltpu.sync_copy(z_hbm.at[pl.ds(off, CH)], z_v)
        gather.wait()

        def row(r, _):
            for j in range(LATENT // L):
                s = pl.ds(j * L, L)
                rows_v[r, s] = rows_v[r, s] * z_v[r, s]
            return 0

        lax.fori_loop(0, CH, row, 0)
        pltpu.sync_copy(rows_v, out_hbm.at[pl.ds(off, CH)])


def kernel(z, label, table):
    return _emb_mul(z, label.astype(jnp.int32), table)


# speedup vs baseline: 1.9490x; 1.0783x over previous
import functools

import jax
import jax.numpy as jnp
from jax import lax
from jax.experimental import pallas as pl
from jax.experimental.pallas import tpu as pltpu
from jax.experimental.pallas import tpu_sc as plsc

LATENT = 128
BATCH = 16384
NC, NS, L = 2, 16, 16
NW = NC * NS
BPW = BATCH // NW
CH = 128
NCHUNK = BPW // CH

_mesh = plsc.VectorSubcoreMesh(core_axis_name="c", subcore_axis_name="s")


@functools.partial(
    pl.kernel,
    mesh=_mesh,
    out_type=jax.ShapeDtypeStruct((BATCH, LATENT), jnp.float32),
    scratch_types=[
        pltpu.VMEM((BPW,), jnp.int32),
        pltpu.VMEM((CH, LATENT), jnp.float32),
        pltpu.VMEM((CH, LATENT), jnp.float32),
        pltpu.VMEM((CH, LATENT), jnp.float32),
        pltpu.VMEM((CH, LATENT), jnp.float32),
        pltpu.SemaphoreType.DMA,
        pltpu.SemaphoreType.DMA,
        pltpu.SemaphoreType.DMA,
        pltpu.SemaphoreType.DMA,
        pltpu.SemaphoreType.DMA,
        pltpu.SemaphoreType.DMA,
    ],
)
def _emb_mul(z_hbm, label_hbm, table_hbm, out_hbm, idx_v,
             z0, z1, r0, r1, sg0, sg1, sz0, sz1, so0, so1):
    wid = lax.axis_index("s") * NC + lax.axis_index("c")
    base = wid * BPW
    zbuf, rbuf = (z0, z1), (r0, r1)
    sg, sz, so = (sg0, sg1), (sz0, sz1), (so0, so1)
    pltpu.sync_copy(label_hbm.at[pl.ds(base, BPW)], idx_v)

    def start(c):
        b = c % 2
        g = pltpu.async_copy(
            table_hbm.at[idx_v.at[pl.ds(c * CH, CH)]], rbuf[b], sg[b])
        zc = pltpu.async_copy(
            z_hbm.at[pl.ds(base + c * CH, CH)], zbuf[b], sz[b])
        return g, zc

    inflight = [None] * NCHUNK
    out_cp = [None] * NCHUNK
    inflight[0] = start(0)
    for c in range(NCHUNK):
        b = c % 2
        if c + 1 < NCHUNK:
            if c >= 1:
                out_cp[c - 1].wait()
            inflight[c + 1] = start(c + 1)
        g, zc = inflight[c]
        g.wait()
        zc.wait()

        def row(r, _):
            for j in range(LATENT // L):
                s = pl.ds(j * L, L)
                rbuf[b][r, s] = rbuf[b][r, s] * zbuf[b][r, s]
            return 0

        lax.fori_loop(0, CH, row, 0)
        out_cp[c] = pltpu.async_copy(
            rbuf[b], out_hbm.at[pl.ds(base + c * CH, CH)], so[b])
    out_cp[NCHUNK - 2].wait()
    out_cp[NCHUNK - 1].wait()


def kernel(z, label, table):
    return _emb_mul(z, label.astype(jnp.int32), table)
